# Initial kernel scaffold; baseline (speedup 1.0000x reference)
import jax
import jax.numpy as jnp
from jax.experimental import pallas as pl


def kernel(x, edge_index, W1, a_src1, a_dst1, b1, W2, a_src2, a_dst2, b2, W3, a_src3, a_dst3, b3):
    raise NotImplementedError("write your pallas kernel here")


# speedup vs baseline: 1.0165x; 1.0165x over previous
import functools

import jax
import jax.numpy as jnp
from jax.experimental import pallas as pl
from jax.experimental.pallas import tpu as pltpu

N = 10000
E = 320000
IN = 128
HEADS = 8
HID = 16
OUT = 64


def _matmul_body(x_ref, w_ref, o_ref):
    o_ref[...] = jnp.dot(x_ref[...], w_ref[...],
                         preferred_element_type=jnp.float32)


def _pallas_matmul(x, w):
    n, k = x.shape
    m = w.shape[1]
    blk = 2000
    return pl.pallas_call(
        _matmul_body,
        grid=(n // blk,),
        in_specs=[
            pl.BlockSpec((blk, k), lambda i: (i, 0)),
            pl.BlockSpec((k, m), lambda i: (0, 0)),
        ],
        out_specs=pl.BlockSpec((blk, m), lambda i: (i, 0)),
        out_shape=jax.ShapeDtypeStruct((n, m), jnp.float32),
    )(x, w)


def _gat_layer(x, src, dst, W, a_s, a_d, b, heads, ch, concat):
    n = x.shape[0]
    h = _pallas_matmul(x, W).reshape(n, heads, ch)
    alpha_src = jnp.sum(h * a_s, axis=-1)
    alpha_dst = jnp.sum(h * a_d, axis=-1)
    e = jax.nn.leaky_relu(alpha_src[src] + alpha_dst[dst], 0.2)
    m = jax.ops.segment_max(e, dst, num_segments=n)
    ex = jnp.exp(e - m[dst])
    den = jax.ops.segment_sum(ex, dst, num_segments=n)
    a = ex / (den[dst] + 1e-16)
    out = jax.ops.segment_sum(h[src] * a[:, :, None], dst, num_segments=n)
    if concat:
        out = out.reshape(n, heads * ch)
    else:
        out = out.mean(axis=1)
    return out + b


def kernel(x, edge_index, W1, a_src1, a_dst1, b1, W2, a_src2, a_dst2, b2,
           W3, a_src3, a_dst3, b3):
    n = x.shape[0]
    loop = jnp.arange(n, dtype=edge_index.dtype)
    src = jnp.concatenate([edge_index[0], loop])
    dst = jnp.concatenate([edge_index[1], loop])
    h = jax.nn.elu(_gat_layer(x, src, dst, W1, a_src1, a_dst1, b1, HEADS, HID, True))
    h = jax.nn.elu(_gat_layer(h, src, dst, W2, a_src2, a_dst2, b2, HEADS, HID, True))
    return _gat_layer(h, src, dst, W3, a_src3, a_dst3, b3, 1, OUT, False)


# speedup vs baseline: 51.7438x; 50.9049x over previous
import functools

import jax
import jax.numpy as jnp
from jax import lax
from jax.experimental import pallas as pl
from jax.experimental.pallas import tpu as pltpu
from jax.experimental.pallas import tpu_sc as plsc

N = 10000
E = 320000
IN = 128
HEADS = 8
HID = 16
OUT = 64

NC = 2
NS = 16
EPT = E // (NC * NS)
B = 80
NCHUNK = EPT // B
ZR = 1000

_GDN = lax.GatherDimensionNumbers(
    offset_dims=(), collapsed_slice_dims=(0,), start_index_map=(0,))


def _vperm(v, idx):
    return lax.gather(v, idx[:, None], _GDN, (1,),
                      mode=lax.GatherScatterMode.PROMISE_IN_BOUNDS)


def _make_sc_edge(C, heads):
    mesh = plsc.VectorSubcoreMesh(core_axis_name="c", subcore_axis_name="s")
    nj = C // 16

    def body(src_hbm, dst_hbm, h_hbm, al_hbm, sh_hbm, zc_hbm, z16_hbm,
             acc0, acc1, den0, den1,
             idx_s, idx_d, ag, bg, pb, hg, shv, sem, acc_sh, den_sh):
        c = lax.axis_index("c")
        s = lax.axis_index("s")
        wid = c * NS + s

        @pl.when(s < 10)
        def _():
            pltpu.sync_copy(zc_hbm, acc_sh.at[pl.ds(s * ZR, ZR)])

        @pl.when(s == 10)
        def _():
            def zden(i, _):
                pltpu.sync_copy(z16_hbm, den_sh.at[pl.ds(i * ZR, ZR)])
                return 0
            lax.fori_loop(0, 10, zden, 0)

        pltpu.sync_copy(sh_hbm, shv)
        plsc.subcore_barrier()

        base = wid * EPT
        sh = shv[...]
        lane = lax.iota(jnp.int32, 16)
        src_pat = lax.rem(lane, 8)
        dst_pat = src_pat + 8

        def chunk(g, _):
            off = base + g * B
            pltpu.sync_copy(src_hbm.at[pl.ds(off, B)], idx_s)
            pltpu.sync_copy(dst_hbm.at[pl.ds(off, B)], idx_d)
            ca = pltpu.async_copy(al_hbm.at[idx_s], ag, sem)
            cb = pltpu.async_copy(al_hbm.at[idx_d], bg, sem)
            cc = pltpu.async_copy(h_hbm.at[idx_s], hg, sem)
            ca.wait()
            cb.wait()
            cc.wait()

            def estep(i, _):
                zs = _vperm(ag[i, :], src_pat)
                zd = _vperm(bg[i, :], dst_pat)
                z = zs + zd
                z = jnp.maximum(z, 0.2 * z)
                p = jnp.exp(z - sh)
                pb[i, :] = p
                for j in range(nj):
                    bc = _vperm(p, jnp.full((16,), j % heads, jnp.int32))
                    hg[i, pl.ds(16 * j, 16)] = hg[i, pl.ds(16 * j, 16)] * bc
                return 0
            lax.fori_loop(0, B, estep, 0)

            pltpu.sync_copy(pb, den_sh.at[idx_d], add=True)
            pltpu.sync_copy(hg, acc_sh.at[idx_d], add=True)
            return 0
        lax.fori_loop(0, NCHUNK, chunk, 0)

        plsc.subcore_barrier()

        @pl.when(s < 10)
        def _():
            r0 = s * ZR

            @pl.when(c == 0)
            def _():
                pltpu.sync_copy(acc_sh.at[pl.ds(r0, ZR)], acc0.at[pl.ds(r0, ZR)])
                pltpu.sync_copy(den_sh.at[pl.ds(r0, ZR)], den0.at[pl.ds(r0, ZR)])

            @pl.when(c == 1)
            def _():
                pltpu.sync_copy(acc_sh.at[pl.ds(r0, ZR)], acc1.at[pl.ds(r0, ZR)])
                pltpu.sync_copy(den_sh.at[pl.ds(r0, ZR)], den1.at[pl.ds(r0, ZR)])

    return pl.kernel(
        body,
        out_type=[
            jax.ShapeDtypeStruct((N, C), jnp.float32),
            jax.ShapeDtypeStruct((N, C), jnp.float32),
            jax.ShapeDtypeStruct((N, 16), jnp.float32),
            jax.ShapeDtypeStruct((N, 16), jnp.float32),
        ],
        mesh=mesh,
        scratch_types=[
            pltpu.VMEM((B,), jnp.int32),
            pltpu.VMEM((B,), jnp.int32),
            pltpu.VMEM((B, 16), jnp.float32),
            pltpu.VMEM((B, 16), jnp.float32),
            pltpu.VMEM((B, 16), jnp.float32),
            pltpu.VMEM((B, C), jnp.float32),
            pltpu.VMEM((16,), jnp.float32),
            pltpu.SemaphoreType.DMA,
            pltpu.VMEM_SHARED((N, C), jnp.float32),
            pltpu.VMEM_SHARED((N, 16), jnp.float32),
        ],
        compiler_params=pltpu.CompilerParams(use_tc_tiling_on_sc=False),
    )


_BLK = 2000
_GRID = N // _BLK


def _maxacc(i, val, m_ref):
    bm = jnp.max(val, axis=0, keepdims=True)

    @pl.when(i == 0)
    def _():
        m_ref[...] = bm

    @pl.when(i > 0)
    def _():
        m_ref[...] = jnp.maximum(m_ref[...], bm)


def _tc_first_body(x_ref, w_ref, ac_ref, h_ref, cat_ref, mc_ref):
    i = pl.program_id(0)
    h = jnp.dot(x_ref[...], w_ref[...], preferred_element_type=jnp.float32)
    h_ref[...] = h
    cat = jnp.dot(h, ac_ref[...], preferred_element_type=jnp.float32)
    cat_ref[...] = cat
    _maxacc(i, cat, mc_ref)


def _tc_first(Cin, Cout):
    return pl.pallas_call(
        _tc_first_body,
        grid=(_GRID,),
        in_specs=[
            pl.BlockSpec((_BLK, Cin), lambda i: (i, 0)),
            pl.BlockSpec((Cin, Cout), lambda i: (0, 0)),
            pl.BlockSpec((Cout, 16), lambda i: (0, 0)),
        ],
        out_specs=[
            pl.BlockSpec((_BLK, Cout), lambda i: (i, 0)),
            pl.BlockSpec((_BLK, 16), lambda i: (i, 0)),
            pl.BlockSpec((1, 16), lambda i: (0, 0)),
        ],
        out_shape=[
            jax.ShapeDtypeStruct((N, Cout), jnp.float32),
            jax.ShapeDtypeStruct((N, 16), jnp.float32),
            jax.ShapeDtypeStruct((1, 16), jnp.float32),
        ],
    )


def _combine(a0, a1, d0, d1, hp, cat, perm, sh, bvec, rm):
    z = cat + jnp.dot(cat, perm, preferred_element_type=jnp.float32)
    pself = jnp.exp(jnp.maximum(z, 0.2 * z) - sh)
    num = a0 + a1 + jnp.dot(pself, rm, preferred_element_type=jnp.float32) * hp
    den = jnp.dot(d0 + d1 + pself, rm, preferred_element_type=jnp.float32)
    return num / (den + 1e-16) + bvec


def _tc_mid_body(a0_ref, a1_ref, d0_ref, d1_ref, hp_ref, cp_ref, perm_ref,
                 sh_ref, b_ref, r_ref, w_ref, ac_ref,
                 h_ref, cat_ref, mc_ref):
    i = pl.program_id(0)
    xn = _combine(a0_ref[...], a1_ref[...], d0_ref[...], d1_ref[...],
                  hp_ref[...], cp_ref[...], perm_ref[...], sh_ref[...],
                  b_ref[...], r_ref[...])
    xn = jnp.where(xn > 0, xn, jnp.exp(jnp.minimum(xn, 0.0)) - 1.0)
    h = jnp.dot(xn, w_ref[...], preferred_element_type=jnp.float32)
    h_ref[...] = h
    cat = jnp.dot(h, ac_ref[...], preferred_element_type=jnp.float32)
    cat_ref[...] = cat
    _maxacc(i, cat, mc_ref)


def _tc_mid(Cin, Cout):
    return pl.pallas_call(
        _tc_mid_body,
        grid=(_GRID,),
        in_specs=[
            pl.BlockSpec((_BLK, Cin), lambda i: (i, 0)),
            pl.BlockSpec((_BLK, Cin), lambda i: (i, 0)),
            pl.BlockSpec((_BLK, 16), lambda i: (i, 0)),
            pl.BlockSpec((_BLK, 16), lambda i: (i, 0)),
            pl.BlockSpec((_BLK, Cin), lambda i: (i, 0)),
            pl.BlockSpec((_BLK, 16), lambda i: (i, 0)),
            pl.BlockSpec((16, 16), lambda i: (0, 0)),
            pl.BlockSpec((1, 16), lambda i: (0, 0)),
            pl.BlockSpec((1, Cin), lambda i: (0, 0)),
            pl.BlockSpec((16, Cin), lambda i: (0, 0)),
            pl.BlockSpec((Cin, Cout), lambda i: (0, 0)),
            pl.BlockSpec((Cout, 16), lambda i: (0, 0)),
        ],
        out_specs=[
            pl.BlockSpec((_BLK, Cout), lambda i: (i, 0)),
            pl.BlockSpec((_BLK, 16), lambda i: (i, 0)),
            pl.BlockSpec((1, 16), lambda i: (0, 0)),
        ],
        out_shape=[
            jax.ShapeDtypeStruct((N, Cout), jnp.float32),
            jax.ShapeDtypeStruct((N, 16), jnp.float32),
            jax.ShapeDtypeStruct((1, 16), jnp.float32),
        ],
    )


def _tc_final_body(a0_ref, a1_ref, d0_ref, d1_ref, hp_ref, cp_ref, perm_ref,
                   sh_ref, b_ref, r_ref, o_ref):
    o_ref[...] = _combine(a0_ref[...], a1_ref[...], d0_ref[...], d1_ref[...],
                          hp_ref[...], cp_ref[...], perm_ref[...], sh_ref[...],
                          b_ref[...], r_ref[...])


def _tc_final(Cin):
    return pl.pallas_call(
        _tc_final_body,
        grid=(_GRID,),
        in_specs=[
            pl.BlockSpec((_BLK, Cin), lambda i: (i, 0)),
            pl.BlockSpec((_BLK, Cin), lambda i: (i, 0)),
            pl.BlockSpec((_BLK, 16), lambda i: (i, 0)),
            pl.BlockSpec((_BLK, 16), lambda i: (i, 0)),
            pl.BlockSpec((_BLK, Cin), lambda i: (i, 0)),
            pl.BlockSpec((_BLK, 16), lambda i: (i, 0)),
            pl.BlockSpec((16, 16), lambda i: (0, 0)),
            pl.BlockSpec((1, 16), lambda i: (0, 0)),
            pl.BlockSpec((1, Cin), lambda i: (0, 0)),
            pl.BlockSpec((16, Cin), lambda i: (0, 0)),
        ],
        out_specs=pl.BlockSpec((_BLK, Cin), lambda i: (i, 0)),
        out_shape=jax.ShapeDtypeStruct((N, Cin), jnp.float32),
    )


def _acat(a_s, a_d):
    hds, ch = a_s.shape
    m = hds * ch
    rows = jnp.arange(m)
    out = jnp.zeros((m, 16), jnp.float32)
    out = out.at[rows, rows // ch].set(a_s.reshape(-1))
    out = out.at[rows, 8 + rows // ch].set(a_d.reshape(-1))
    return out


def _shifts(mcat):
    s8 = jnp.maximum(mcat[0, :8] + mcat[0, 8:], 0.0)
    s16 = jnp.concatenate([s8, s8])
    return s16.reshape(1, 16), s16


def kernel(x, edge_index, W1, a_src1, a_dst1, b1, W2, a_src2, a_dst2, b2,
           W3, a_src3, a_dst3, b3):
    src = edge_index[0]
    dst = edge_index[1]

    m = HEADS * HID
    ac1 = _acat(a_src1, a_dst1)
    ac2 = _acat(a_src2, a_dst2)
    ac3 = _acat(a_src3.reshape(1, OUT), a_dst3.reshape(1, OUT))

    cols = jnp.arange(m)
    r8 = jnp.zeros((16, m), jnp.float32).at[cols // HID, cols].set(1.0)
    r3 = jnp.zeros((16, OUT), jnp.float32).at[0, :].set(1.0)
    ii = jnp.arange(16)
    perm = jnp.zeros((16, 16), jnp.float32).at[ii, (ii + 8) % 16].set(1.0)

    zc128 = jnp.zeros((ZR, m), jnp.float32)
    zc64 = jnp.zeros((ZR, OUT), jnp.float32)
    z16 = jnp.zeros((ZR, 16), jnp.float32)

    sc128 = _make_sc_edge(m, HEADS)
    sc64 = _make_sc_edge(OUT, 1)

    h1, c1, mc1 = _tc_first(IN, m)(x, W1, ac1)
    sh1, shv1 = _shifts(mc1)
    a0, a1_, dn0, dn1 = sc128(src, dst, h1, c1, shv1, zc128, z16)

    h2, c2, mc2 = _tc_mid(m, m)(
        a0, a1_, dn0, dn1, h1, c1, perm, sh1, b1.reshape(1, -1), r8, W2, ac2)
    sh2, shv2 = _shifts(mc2)
    a0, a1_, dn0, dn1 = sc128(src, dst, h2, c2, shv2, zc128, z16)

    h3, c3, mc3 = _tc_mid(m, OUT)(
        a0, a1_, dn0, dn1, h2, c2, perm, sh2, b2.reshape(1, -1), r8, W3, ac3)
    sh3, shv3 = _shifts(mc3)
    a0, a1_, dn0, dn1 = sc64(src, dst, h3, c3, shv3, zc64, z16)

    return _tc_final(OUT)(a0, a1_, dn0, dn1, h3, c3, perm, sh3,
                          b3.reshape(1, -1), r3)


# speedup vs baseline: 71.2395x; 1.3768x over previous
import functools

import jax
import jax.numpy as jnp
from jax import lax
from jax.experimental import pallas as pl
from jax.experimental.pallas import tpu as pltpu
from jax.experimental.pallas import tpu_sc as plsc

N = 10000
E = 320000
IN = 128
HEADS = 8
HID = 16
OUT = 64

NC = 2
NS = 16
EPT = E // (NC * NS)
B = 80
NCHUNK = EPT // B
ZR = 1000

_GDN = lax.GatherDimensionNumbers(
    offset_dims=(), collapsed_slice_dims=(0,), start_index_map=(0,))


def _vperm(v, idx):
    return lax.gather(v, idx[:, None], _GDN, (1,),
                      mode=lax.GatherScatterMode.PROMISE_IN_BOUNDS)


def _make_sc_edge(C, heads):
    mesh = plsc.VectorSubcoreMesh(core_axis_name="c", subcore_axis_name="s")
    nj = C // 16

    def body(src_hbm, dst_hbm, h_hbm, al_hbm, sh_hbm, zc_hbm, z16_hbm,
             acc0, acc1, den0, den1,
             idx_s0, idx_d0, ag0, bg0, pb0, hg0,
             idx_s1, idx_d1, ag1, bg1, pb1, hg1,
             shv, gsem0, gsem1, ssem0, ssem1, acc_sh, den_sh):
        c = lax.axis_index("c")
        s = lax.axis_index("s")
        wid = c * NS + s

        @pl.when(s < 10)
        def _():
            pltpu.sync_copy(zc_hbm, acc_sh.at[pl.ds(s * ZR, ZR)])

        @pl.when(s == 10)
        def _():
            def zden(i, _):
                pltpu.sync_copy(z16_hbm, den_sh.at[pl.ds(i * ZR, ZR)])
                return 0
            lax.fori_loop(0, 10, zden, 0)

        pltpu.sync_copy(sh_hbm, shv)
        plsc.subcore_barrier()

        base = wid * EPT
        sh = shv[...]
        lane = lax.iota(jnp.int32, 16)
        src_pat = lax.rem(lane, 8)
        dst_pat = src_pat + 8

        sets = ((idx_s0, idx_d0, ag0, bg0, pb0, hg0, gsem0, ssem0),
                (idx_s1, idx_d1, ag1, bg1, pb1, hg1, gsem1, ssem1))

        def fire(bi, g):
            iS, iD, A, Bg, P, H, gs, ss = sets[bi]
            off = base + g * B
            pltpu.sync_copy(src_hbm.at[pl.ds(off, B)], iS)
            pltpu.sync_copy(dst_hbm.at[pl.ds(off, B)], iD)
            pltpu.async_copy(al_hbm.at[iS], A, gs)
            pltpu.async_copy(al_hbm.at[iD], Bg, gs)
            pltpu.async_copy(h_hbm.at[iS], H, gs)

        def drain_gather(bi):
            iS, iD, A, Bg, P, H, gs, ss = sets[bi]
            pltpu.make_async_copy(al_hbm.at[iS], A, gs).wait()
            pltpu.make_async_copy(al_hbm.at[iD], Bg, gs).wait()
            pltpu.make_async_copy(h_hbm.at[iS], H, gs).wait()

        def compute(bi):
            iS, iD, A, Bg, P, H, gs, ss = sets[bi]

            def estep(i, _):
                zs = _vperm(A[i, :], src_pat)
                zd = _vperm(Bg[i, :], dst_pat)
                z = zs + zd
                z = jnp.maximum(z, 0.2 * z)
                p = jnp.exp(z - sh)
                P[i, :] = p
                for j in range(nj):
                    bc = _vperm(p, jnp.full((16,), j % heads, jnp.int32))
                    H[i, pl.ds(16 * j, 16)] = H[i, pl.ds(16 * j, 16)] * bc
                return 0
            lax.fori_loop(0, B, estep, 0)

        def scatter(bi):
            iS, iD, A, Bg, P, H, gs, ss = sets[bi]
            pltpu.async_copy(P, den_sh.at[iD], ss, add=True)
            pltpu.async_copy(H, acc_sh.at[iD], ss, add=True)

        def drain_scatter(bi):
            iS, iD, A, Bg, P, H, gs, ss = sets[bi]
            pltpu.make_async_copy(P, den_sh.at[iD], ss).wait()
            pltpu.make_async_copy(H, acc_sh.at[iD], ss).wait()

        fire(0, 0)
        fire(1, 1)
        drain_gather(0)
        compute(0)
        scatter(0)
        drain_gather(1)
        compute(1)
        drain_scatter(0)
        fire(0, 2)
        scatter(1)

        def pipe(k, _):
            drain_scatter(1)
            fire(1, 2 * k + 1)
            drain_gather(0)
            compute(0)
            scatter(0)
            drain_gather(1)
            compute(1)
            drain_scatter(0)
            fire(0, 2 * k + 2)
            scatter(1)
            return 0
        lax.fori_loop(1, (NCHUNK - 1) // 2, pipe, 0)

        drain_gather(0)
        compute(0)
        scatter(0)
        drain_scatter(1)
        drain_scatter(0)

        plsc.subcore_barrier()

        @pl.when(s < 10)
        def _():
            r0 = s * ZR

            @pl.when(c == 0)
            def _():
                pltpu.sync_copy(acc_sh.at[pl.ds(r0, ZR)], acc0.at[pl.ds(r0, ZR)])
                pltpu.sync_copy(den_sh.at[pl.ds(r0, ZR)], den0.at[pl.ds(r0, ZR)])

            @pl.when(c == 1)
            def _():
                pltpu.sync_copy(acc_sh.at[pl.ds(r0, ZR)], acc1.at[pl.ds(r0, ZR)])
                pltpu.sync_copy(den_sh.at[pl.ds(r0, ZR)], den1.at[pl.ds(r0, ZR)])

    return pl.kernel(
        body,
        out_type=[
            jax.ShapeDtypeStruct((N, C), jnp.float32),
            jax.ShapeDtypeStruct((N, C), jnp.float32),
            jax.ShapeDtypeStruct((N, 16), jnp.float32),
            jax.ShapeDtypeStruct((N, 16), jnp.float32),
        ],
        mesh=mesh,
        scratch_types=(
            [
                pltpu.VMEM((B,), jnp.int32),
                pltpu.VMEM((B,), jnp.int32),
                pltpu.VMEM((B, 16), jnp.float32),
                pltpu.VMEM((B, 16), jnp.float32),
                pltpu.VMEM((B, 16), jnp.float32),
                pltpu.VMEM((B, C), jnp.float32),
            ] * 2
            + [
                pltpu.VMEM((16,), jnp.float32),
                pltpu.SemaphoreType.DMA,
                pltpu.SemaphoreType.DMA,
                pltpu.SemaphoreType.DMA,
                pltpu.SemaphoreType.DMA,
                pltpu.VMEM_SHARED((N, C), jnp.float32),
                pltpu.VMEM_SHARED((N, 16), jnp.float32),
            ]
        ),
        compiler_params=pltpu.CompilerParams(use_tc_tiling_on_sc=False),
    )


_BLK = 2000
_GRID = N // _BLK


def _maxacc(i, val, m_ref):
    bm = jnp.max(val, axis=0, keepdims=True)

    @pl.when(i == 0)
    def _():
        m_ref[...] = bm

    @pl.when(i > 0)
    def _():
        m_ref[...] = jnp.maximum(m_ref[...], bm)


def _tc_first_body(x_ref, w_ref, ac_ref, h_ref, cat_ref, mc_ref):
    i = pl.program_id(0)
    h = jnp.dot(x_ref[...], w_ref[...], preferred_element_type=jnp.float32)
    h_ref[...] = h
    cat = jnp.dot(h, ac_ref[...], preferred_element_type=jnp.float32)
    cat_ref[...] = cat
    _maxacc(i, cat, mc_ref)


def _tc_first(Cin, Cout):
    return pl.pallas_call(
        _tc_first_body,
        grid=(_GRID,),
        in_specs=[
            pl.BlockSpec((_BLK, Cin), lambda i: (i, 0)),
            pl.BlockSpec((Cin, Cout), lambda i: (0, 0)),
            pl.BlockSpec((Cout, 16), lambda i: (0, 0)),
        ],
        out_specs=[
            pl.BlockSpec((_BLK, Cout), lambda i: (i, 0)),
            pl.BlockSpec((_BLK, 16), lambda i: (i, 0)),
            pl.BlockSpec((1, 16), lambda i: (0, 0)),
        ],
        out_shape=[
            jax.ShapeDtypeStruct((N, Cout), jnp.float32),
            jax.ShapeDtypeStruct((N, 16), jnp.float32),
            jax.ShapeDtypeStruct((1, 16), jnp.float32),
        ],
    )


def _combine(a0, a1, d0, d1, hp, cat, perm, sh, bvec, rm):
    z = cat + jnp.dot(cat, perm, preferred_element_type=jnp.float32)
    pself = jnp.exp(jnp.maximum(z, 0.2 * z) - sh)
    num = a0 + a1 + jnp.dot(pself, rm, preferred_element_type=jnp.float32) * hp
    den = jnp.dot(d0 + d1 + pself, rm, preferred_element_type=jnp.float32)
    return num / (den + 1e-16) + bvec


def _tc_mid_body(a0_ref, a1_ref, d0_ref, d1_ref, hp_ref, cp_ref, perm_ref,
                 sh_ref, b_ref, r_ref, w_ref, ac_ref,
                 h_ref, cat_ref, mc_ref):
    i = pl.program_id(0)
    xn = _combine(a0_ref[...], a1_ref[...], d0_ref[...], d1_ref[...],
                  hp_ref[...], cp_ref[...], perm_ref[...], sh_ref[...],
                  b_ref[...], r_ref[...])
    xn = jnp.where(xn > 0, xn, jnp.exp(jnp.minimum(xn, 0.0)) - 1.0)
    h = jnp.dot(xn, w_ref[...], preferred_element_type=jnp.float32)
    h_ref[...] = h
    cat = jnp.dot(h, ac_ref[...], preferred_element_type=jnp.float32)
    cat_ref[...] = cat
    _maxacc(i, cat, mc_ref)


def _tc_mid(Cin, Cout):
    return pl.pallas_call(
        _tc_mid_body,
        grid=(_GRID,),
        in_specs=[
            pl.BlockSpec((_BLK, Cin), lambda i: (i, 0)),
            pl.BlockSpec((_BLK, Cin), lambda i: (i, 0)),
            pl.BlockSpec((_BLK, 16), lambda i: (i, 0)),
            pl.BlockSpec((_BLK, 16), lambda i: (i, 0)),
            pl.BlockSpec((_BLK, Cin), lambda i: (i, 0)),
            pl.BlockSpec((_BLK, 16), lambda i: (i, 0)),
            pl.BlockSpec((16, 16), lambda i: (0, 0)),
            pl.BlockSpec((1, 16), lambda i: (0, 0)),
            pl.BlockSpec((1, Cin), lambda i: (0, 0)),
            pl.BlockSpec((16, Cin), lambda i: (0, 0)),
            pl.BlockSpec((Cin, Cout), lambda i: (0, 0)),
            pl.BlockSpec((Cout, 16), lambda i: (0, 0)),
        ],
        out_specs=[
            pl.BlockSpec((_BLK, Cout), lambda i: (i, 0)),
            pl.BlockSpec((_BLK, 16), lambda i: (i, 0)),
            pl.BlockSpec((1, 16), lambda i: (0, 0)),
        ],
        out_shape=[
            jax.ShapeDtypeStruct((N, Cout), jnp.float32),
            jax.ShapeDtypeStruct((N, 16), jnp.float32),
            jax.ShapeDtypeStruct((1, 16), jnp.float32),
        ],
    )


def _tc_final_body(a0_ref, a1_ref, d0_ref, d1_ref, hp_ref, cp_ref, perm_ref,
                   sh_ref, b_ref, r_ref, o_ref):
    o_ref[...] = _combine(a0_ref[...], a1_ref[...], d0_ref[...], d1_ref[...],
                          hp_ref[...], cp_ref[...], perm_ref[...], sh_ref[...],
                          b_ref[...], r_ref[...])


def _tc_final(Cin):
    return pl.pallas_call(
        _tc_final_body,
        grid=(_GRID,),
        in_specs=[
            pl.BlockSpec((_BLK, Cin), lambda i: (i, 0)),
            pl.BlockSpec((_BLK, Cin), lambda i: (i, 0)),
            pl.BlockSpec((_BLK, 16), lambda i: (i, 0)),
            pl.BlockSpec((_BLK, 16), lambda i: (i, 0)),
            pl.BlockSpec((_BLK, Cin), lambda i: (i, 0)),
            pl.BlockSpec((_BLK, 16), lambda i: (i, 0)),
            pl.BlockSpec((16, 16), lambda i: (0, 0)),
            pl.BlockSpec((1, 16), lambda i: (0, 0)),
            pl.BlockSpec((1, Cin), lambda i: (0, 0)),
            pl.BlockSpec((16, Cin), lambda i: (0, 0)),
        ],
        out_specs=pl.BlockSpec((_BLK, Cin), lambda i: (i, 0)),
        out_shape=jax.ShapeDtypeStruct((N, Cin), jnp.float32),
    )


def _acat(a_s, a_d):
    hds, ch = a_s.shape
    m = hds * ch
    rows = jnp.arange(m)
    out = jnp.zeros((m, 16), jnp.float32)
    out = out.at[rows, rows // ch].set(a_s.reshape(-1))
    out = out.at[rows, 8 + rows // ch].set(a_d.reshape(-1))
    return out


def _shifts(mcat):
    s8 = jnp.maximum(mcat[0, :8] + mcat[0, 8:], 0.0)
    s16 = jnp.concatenate([s8, s8])
    return s16.reshape(1, 16), s16


def kernel(x, edge_index, W1, a_src1, a_dst1, b1, W2, a_src2, a_dst2, b2,
           W3, a_src3, a_dst3, b3):
    src = edge_index[0]
    dst = edge_index[1]

    m = HEADS * HID
    ac1 = _acat(a_src1, a_dst1)
    ac2 = _acat(a_src2, a_dst2)
    ac3 = _acat(a_src3.reshape(1, OUT), a_dst3.reshape(1, OUT))

    cols = jnp.arange(m)
    r8 = jnp.zeros((16, m), jnp.float32).at[cols // HID, cols].set(1.0)
    r3 = jnp.zeros((16, OUT), jnp.float32).at[0, :].set(1.0)
    ii = jnp.arange(16)
    perm = jnp.zeros((16, 16), jnp.float32).at[ii, (ii + 8) % 16].set(1.0)

    zc128 = jnp.zeros((ZR, m), jnp.float32)
    zc64 = jnp.zeros((ZR, OUT), jnp.float32)
    z16 = jnp.zeros((ZR, 16), jnp.float32)

    sc128 = _make_sc_edge(m, HEADS)
    sc64 = _make_sc_edge(OUT, 1)

    h1, c1, mc1 = _tc_first(IN, m)(x, W1, ac1)
    sh1, shv1 = _shifts(mc1)
    a0, a1_, dn0, dn1 = sc128(src, dst, h1, c1, shv1, zc128, z16)

    h2, c2, mc2 = _tc_mid(m, m)(
        a0, a1_, dn0, dn1, h1, c1, perm, sh1, b1.reshape(1, -1), r8, W2, ac2)
    sh2, shv2 = _shifts(mc2)
    a0, a1_, dn0, dn1 = sc128(src, dst, h2, c2, shv2, zc128, z16)

    h3, c3, mc3 = _tc_mid(m, OUT)(
        a0, a1_, dn0, dn1, h2, c2, perm, sh2, b2.reshape(1, -1), r8, W3, ac3)
    sh3, shv3 = _shifts(mc3)
    a0, a1_, dn0, dn1 = sc64(src, dst, h3, c3, shv3, zc64, z16)

    return _tc_final(OUT)(a0, a1_, dn0, dn1, h3, c3, perm, sh3,
                          b3.reshape(1, -1), r3)


# speedup vs baseline: 76.3375x; 1.0716x over previous
import functools

import jax
import jax.numpy as jnp
from jax import lax
from jax.experimental import pallas as pl
from jax.experimental.pallas import tpu as pltpu
from jax.experimental.pallas import tpu_sc as plsc

N = 10000
E = 320000
IN = 128
HEADS = 8
HID = 16
OUT = 64

NC = 2
NS = 16
EPT = E // (NC * NS)
B = 80
NCHUNK = EPT // B
ZR = 1000

_GDN = lax.GatherDimensionNumbers(
    offset_dims=(), collapsed_slice_dims=(0,), start_index_map=(0,))


def _vperm(v, idx):
    return lax.gather(v, idx[:, None], _GDN, (1,),
                      mode=lax.GatherScatterMode.PROMISE_IN_BOUNDS)


def _make_sc_edge(C, heads):
    mesh = plsc.VectorSubcoreMesh(core_axis_name="c", subcore_axis_name="s")
    nj = C // 16

    def body(src_hbm, dst_hbm, h_hbm, al_hbm, sh_hbm, zc_hbm, z16_hbm,
             acc0, acc1, den0, den1,
             idx_s0, idx_d0, ag0, bg0, pb0, hg0,
             idx_s1, idx_d1, ag1, bg1, pb1, hg1,
             shv, gsem0, gsem1, ssem0, ssem1, isem0, isem1,
             acc_sh, den_sh):
        c = lax.axis_index("c")
        s = lax.axis_index("s")
        wid = c * NS + s

        @pl.when(s < 10)
        def _():
            pltpu.sync_copy(zc_hbm, acc_sh.at[pl.ds(s * ZR, ZR)])

        @pl.when(s == 10)
        def _():
            def zden(i, _):
                pltpu.sync_copy(z16_hbm, den_sh.at[pl.ds(i * ZR, ZR)])
                return 0
            lax.fori_loop(0, 10, zden, 0)

        pltpu.sync_copy(sh_hbm, shv)
        plsc.subcore_barrier()

        base = wid * EPT
        sh = shv[...]
        lane = lax.iota(jnp.int32, 16)
        src_pat = lax.rem(lane, 8)
        dst_pat = src_pat + 8

        sets = ((idx_s0, idx_d0, ag0, bg0, pb0, hg0, gsem0, ssem0, isem0),
                (idx_s1, idx_d1, ag1, bg1, pb1, hg1, gsem1, ssem1, isem1))

        def fire_idx(bi, g):
            iS, iD, A, Bg, P, H, gs, ss, isem = sets[bi]
            off = base + g * B
            pltpu.async_copy(src_hbm.at[pl.ds(off, B)], iS, isem)
            pltpu.async_copy(dst_hbm.at[pl.ds(off, B)], iD, isem)

        def fire_gather(bi):
            iS, iD, A, Bg, P, H, gs, ss, isem = sets[bi]
            pltpu.make_async_copy(src_hbm.at[pl.ds(base, B)], iS, isem).wait()
            pltpu.make_async_copy(dst_hbm.at[pl.ds(base, B)], iD, isem).wait()
            pltpu.async_copy(al_hbm.at[iS], A, gs)
            pltpu.async_copy(al_hbm.at[iD], Bg, gs)
            pltpu.async_copy(h_hbm.at[iS], H, gs)

        def drain_gather(bi):
            iS, iD, A, Bg, P, H, gs, ss, isem = sets[bi]
            pltpu.make_async_copy(al_hbm.at[iS], A, gs).wait()
            pltpu.make_async_copy(al_hbm.at[iD], Bg, gs).wait()
            pltpu.make_async_copy(h_hbm.at[iS], H, gs).wait()

        def compute(bi):
            iS, iD, A, Bg, P, H, gs, ss, isem = sets[bi]

            def estep(i, _):
                zs = _vperm(A[i, :], src_pat)
                zd = _vperm(Bg[i, :], dst_pat)
                z = zs + zd
                z = jnp.maximum(z, 0.2 * z)
                p = jnp.exp(z - sh)
                P[i, :] = p
                for j in range(nj):
                    bc = _vperm(p, jnp.full((16,), j % heads, jnp.int32))
                    H[i, pl.ds(16 * j, 16)] = H[i, pl.ds(16 * j, 16)] * bc
                return 0
            lax.fori_loop(0, B, estep, 0, unroll=2)

        def scatter(bi):
            iS, iD, A, Bg, P, H, gs, ss, isem = sets[bi]
            pltpu.async_copy(P, den_sh.at[iD], ss, add=True)
            pltpu.async_copy(H, acc_sh.at[iD], ss, add=True)

        def drain_scatter(bi):
            iS, iD, A, Bg, P, H, gs, ss, isem = sets[bi]
            pltpu.make_async_copy(P, den_sh.at[iD], ss).wait()
            pltpu.make_async_copy(H, acc_sh.at[iD], ss).wait()

        fire_idx(0, 0)
        fire_gather(0)
        fire_idx(1, 1)
        fire_gather(1)
        drain_gather(0)
        compute(0)
        scatter(0)
        drain_scatter(0)
        fire_idx(0, 2)
        drain_gather(1)
        fire_gather(0)
        compute(1)
        scatter(1)

        def pipe(k, _):
            drain_scatter(1)
            fire_idx(1, 2 * k + 1)
            drain_gather(0)
            fire_gather(1)
            compute(0)
            scatter(0)
            drain_scatter(0)
            fire_idx(0, 2 * k + 2)
            drain_gather(1)
            fire_gather(0)
            compute(1)
            scatter(1)
            return 0
        lax.fori_loop(1, (NCHUNK - 1) // 2, pipe, 0)

        drain_gather(0)
        compute(0)
        scatter(0)
        drain_scatter(1)
        drain_scatter(0)

        plsc.subcore_barrier()

        @pl.when(s < 10)
        def _():
            r0 = s * ZR

            @pl.when(c == 0)
            def _():
                pltpu.sync_copy(acc_sh.at[pl.ds(r0, ZR)], acc0.at[pl.ds(r0, ZR)])
                pltpu.sync_copy(den_sh.at[pl.ds(r0, ZR)], den0.at[pl.ds(r0, ZR)])

            @pl.when(c == 1)
            def _():
                pltpu.sync_copy(acc_sh.at[pl.ds(r0, ZR)], acc1.at[pl.ds(r0, ZR)])
                pltpu.sync_copy(den_sh.at[pl.ds(r0, ZR)], den1.at[pl.ds(r0, ZR)])

    return pl.kernel(
        body,
        out_type=[
            jax.ShapeDtypeStruct((N, C), jnp.float32),
            jax.ShapeDtypeStruct((N, C), jnp.float32),
            jax.ShapeDtypeStruct((N, 16), jnp.float32),
            jax.ShapeDtypeStruct((N, 16), jnp.float32),
        ],
        mesh=mesh,
        scratch_types=(
            [
                pltpu.VMEM((B,), jnp.int32),
                pltpu.VMEM((B,), jnp.int32),
                pltpu.VMEM((B, 16), jnp.float32),
                pltpu.VMEM((B, 16), jnp.float32),
                pltpu.VMEM((B, 16), jnp.float32),
                pltpu.VMEM((B, C), jnp.float32),
            ] * 2
            + [
                pltpu.VMEM((16,), jnp.float32),
                pltpu.SemaphoreType.DMA,
                pltpu.SemaphoreType.DMA,
                pltpu.SemaphoreType.DMA,
                pltpu.SemaphoreType.DMA,
                pltpu.SemaphoreType.DMA,
                pltpu.SemaphoreType.DMA,
                pltpu.VMEM_SHARED((N, C), jnp.float32),
                pltpu.VMEM_SHARED((N, 16), jnp.float32),
            ]
        ),
        compiler_params=pltpu.CompilerParams(use_tc_tiling_on_sc=False),
    )


_BLK = 2000
_GRID = N // _BLK


def _maxacc(i, val, m_ref):
    bm = jnp.max(val, axis=0, keepdims=True)

    @pl.when(i == 0)
    def _():
        m_ref[...] = bm

    @pl.when(i > 0)
    def _():
        m_ref[...] = jnp.maximum(m_ref[...], bm)


def _tc_first_body(x_ref, w_ref, ac_ref, h_ref, cat_ref, mc_ref):
    i = pl.program_id(0)
    h = jnp.dot(x_ref[...], w_ref[...], preferred_element_type=jnp.float32)
    h_ref[...] = h
    cat = jnp.dot(h, ac_ref[...], preferred_element_type=jnp.float32)
    cat_ref[...] = cat
    _maxacc(i, cat, mc_ref)


def _tc_first(Cin, Cout):
    return pl.pallas_call(
        _tc_first_body,
        grid=(_GRID,),
        in_specs=[
            pl.BlockSpec((_BLK, Cin), lambda i: (i, 0)),
            pl.BlockSpec((Cin, Cout), lambda i: (0, 0)),
            pl.BlockSpec((Cout, 16), lambda i: (0, 0)),
        ],
        out_specs=[
            pl.BlockSpec((_BLK, Cout), lambda i: (i, 0)),
            pl.BlockSpec((_BLK, 16), lambda i: (i, 0)),
            pl.BlockSpec((1, 16), lambda i: (0, 0)),
        ],
        out_shape=[
            jax.ShapeDtypeStruct((N, Cout), jnp.float32),
            jax.ShapeDtypeStruct((N, 16), jnp.float32),
            jax.ShapeDtypeStruct((1, 16), jnp.float32),
        ],
    )


def _combine(a0, a1, d0, d1, hp, cat, perm, sh, bvec, rm):
    z = cat + jnp.dot(cat, perm, preferred_element_type=jnp.float32)
    pself = jnp.exp(jnp.maximum(z, 0.2 * z) - sh)
    num = a0 + a1 + jnp.dot(pself, rm, preferred_element_type=jnp.float32) * hp
    den = jnp.dot(d0 + d1 + pself, rm, preferred_element_type=jnp.float32)
    return num / (den + 1e-16) + bvec


def _tc_mid_body(a0_ref, a1_ref, d0_ref, d1_ref, hp_ref, cp_ref, perm_ref,
                 sh_ref, b_ref, r_ref, w_ref, ac_ref,
                 h_ref, cat_ref, mc_ref):
    i = pl.program_id(0)
    xn = _combine(a0_ref[...], a1_ref[...], d0_ref[...], d1_ref[...],
                  hp_ref[...], cp_ref[...], perm_ref[...], sh_ref[...],
                  b_ref[...], r_ref[...])
    xn = jnp.where(xn > 0, xn, jnp.exp(jnp.minimum(xn, 0.0)) - 1.0)
    h = jnp.dot(xn, w_ref[...], preferred_element_type=jnp.float32)
    h_ref[...] = h
    cat = jnp.dot(h, ac_ref[...], preferred_element_type=jnp.float32)
    cat_ref[...] = cat
    _maxacc(i, cat, mc_ref)


def _tc_mid(Cin, Cout):
    return pl.pallas_call(
        _tc_mid_body,
        grid=(_GRID,),
        in_specs=[
            pl.BlockSpec((_BLK, Cin), lambda i: (i, 0)),
            pl.BlockSpec((_BLK, Cin), lambda i: (i, 0)),
            pl.BlockSpec((_BLK, 16), lambda i: (i, 0)),
            pl.BlockSpec((_BLK, 16), lambda i: (i, 0)),
            pl.BlockSpec((_BLK, Cin), lambda i: (i, 0)),
            pl.BlockSpec((_BLK, 16), lambda i: (i, 0)),
            pl.BlockSpec((16, 16), lambda i: (0, 0)),
            pl.BlockSpec((1, 16), lambda i: (0, 0)),
            pl.BlockSpec((1, Cin), lambda i: (0, 0)),
            pl.BlockSpec((16, Cin), lambda i: (0, 0)),
            pl.BlockSpec((Cin, Cout), lambda i: (0, 0)),
            pl.BlockSpec((Cout, 16), lambda i: (0, 0)),
        ],
        out_specs=[
            pl.BlockSpec((_BLK, Cout), lambda i: (i, 0)),
            pl.BlockSpec((_BLK, 16), lambda i: (i, 0)),
            pl.BlockSpec((1, 16), lambda i: (0, 0)),
        ],
        out_shape=[
            jax.ShapeDtypeStruct((N, Cout), jnp.float32),
            jax.ShapeDtypeStruct((N, 16), jnp.float32),
            jax.ShapeDtypeStruct((1, 16), jnp.float32),
        ],
    )


def _tc_final_body(a0_ref, a1_ref, d0_ref, d1_ref, hp_ref, cp_ref, perm_ref,
                   sh_ref, b_ref, r_ref, o_ref):
    o_ref[...] = _combine(a0_ref[...], a1_ref[...], d0_ref[...], d1_ref[...],
                          hp_ref[...], cp_ref[...], perm_ref[...], sh_ref[...],
                          b_ref[...], r_ref[...])


def _tc_final(Cin):
    return pl.pallas_call(
        _tc_final_body,
        grid=(_GRID,),
        in_specs=[
            pl.BlockSpec((_BLK, Cin), lambda i: (i, 0)),
            pl.BlockSpec((_BLK, Cin), lambda i: (i, 0)),
            pl.BlockSpec((_BLK, 16), lambda i: (i, 0)),
            pl.BlockSpec((_BLK, 16), lambda i: (i, 0)),
            pl.BlockSpec((_BLK, Cin), lambda i: (i, 0)),
            pl.BlockSpec((_BLK, 16), lambda i: (i, 0)),
            pl.BlockSpec((16, 16), lambda i: (0, 0)),
            pl.BlockSpec((1, 16), lambda i: (0, 0)),
            pl.BlockSpec((1, Cin), lambda i: (0, 0)),
            pl.BlockSpec((16, Cin), lambda i: (0, 0)),
        ],
        out_specs=pl.BlockSpec((_BLK, Cin), lambda i: (i, 0)),
        out_shape=jax.ShapeDtypeStruct((N, Cin), jnp.float32),
    )


def _acat(a_s, a_d):
    hds, ch = a_s.shape
    m = hds * ch
    rows = jnp.arange(m)
    out = jnp.zeros((m, 16), jnp.float32)
    out = out.at[rows, rows // ch].set(a_s.reshape(-1))
    out = out.at[rows, 8 + rows // ch].set(a_d.reshape(-1))
    return out


def _shifts(mcat):
    s8 = jnp.maximum(mcat[0, :8] + mcat[0, 8:], 0.0)
    s16 = jnp.concatenate([s8, s8])
    return s16.reshape(1, 16), s16


def kernel(x, edge_index, W1, a_src1, a_dst1, b1, W2, a_src2, a_dst2, b2,
           W3, a_src3, a_dst3, b3):
    src = edge_index[0]
    dst = edge_index[1]

    m = HEADS * HID
    ac1 = _acat(a_src1, a_dst1)
    ac2 = _acat(a_src2, a_dst2)
    ac3 = _acat(a_src3.reshape(1, OUT), a_dst3.reshape(1, OUT))

    cols = jnp.arange(m)
    r8 = jnp.zeros((16, m), jnp.float32).at[cols // HID, cols].set(1.0)
    r3 = jnp.zeros((16, OUT), jnp.float32).at[0, :].set(1.0)
    ii = jnp.arange(16)
    perm = jnp.zeros((16, 16), jnp.float32).at[ii, (ii + 8) % 16].set(1.0)

    zc128 = jnp.zeros((ZR, m), jnp.float32)
    zc64 = jnp.zeros((ZR, OUT), jnp.float32)
    z16 = jnp.zeros((ZR, 16), jnp.float32)

    sc128 = _make_sc_edge(m, HEADS)
    sc64 = _make_sc_edge(OUT, 1)

    h1, c1, mc1 = _tc_first(IN, m)(x, W1, ac1)
    sh1, shv1 = _shifts(mc1)
    a0, a1_, dn0, dn1 = sc128(src, dst, h1, c1, shv1, zc128, z16)

    h2, c2, mc2 = _tc_mid(m, m)(
        a0, a1_, dn0, dn1, h1, c1, perm, sh1, b1.reshape(1, -1), r8, W2, ac2)
    sh2, shv2 = _shifts(mc2)
    a0, a1_, dn0, dn1 = sc128(src, dst, h2, c2, shv2, zc128, z16)

    h3, c3, mc3 = _tc_mid(m, OUT)(
        a0, a1_, dn0, dn1, h2, c2, perm, sh2, b2.reshape(1, -1), r8, W3, ac3)
    sh3, shv3 = _shifts(mc3)
    a0, a1_, dn0, dn1 = sc64(src, dst, h3, c3, shv3, zc64, z16)

    return _tc_final(OUT)(a0, a1_, dn0, dn1, h3, c3, perm, sh3,
                          b3.reshape(1, -1), r3)
